# baseline (speedup 1.0000x reference)
import functools

import jax
import jax.numpy as jnp
from jax import lax
from jax.experimental import pallas as pl
from jax.experimental.pallas import tpu as pltpu
from jax.experimental.pallas import tpu_sc as plsc

NC = 2
NS = 16
NW = NC * NS
LN = 16
EB = 80
PW = 80


def _tc1_body(x_ref, wcat_ref, att_ref, xl_ref, xr_ref, mh_ref):
    y = jnp.dot(x_ref[...], wcat_ref[...], preferred_element_type=jnp.float32)
    d = y.shape[1] // 2
    xl = y[:, :d]
    xr = y[:, d:]
    xl_ref[...] = xl
    xr_ref[...] = xr
    s = xl + xr
    lr = jnp.where(s >= 0, s, 0.2 * s)
    mh_ref[...] = jnp.dot(lr, att_ref[...], preferred_element_type=jnp.float32)


def _tc2_body(acc_ref, xl_ref, bias_ref, wlin_ref, blin_ref, out_ref):
    n = xl_ref.shape[0]
    s = acc_ref[0][:n] + acc_ref[1][:n]
    d = xl_ref.shape[1]
    h = s[:, :d] + xl_ref[...]
    den = s[:, d:d + 1] + 1.0
    h = h / den + bias_ref[...]
    h = jnp.maximum(h, 0.0)
    z = jnp.dot(h, wlin_ref[...], preferred_element_type=jnp.float32)
    out_ref[...] = jax.nn.sigmoid(z + blin_ref[...])


def _sc_body(nb, rps,
             ei_hbm, xl_hbm, xr_hbm, mh_hbm, att_hbm, out_hbm,
             mh_v, att_v, src_all, dst_all, xl0, xl1, xr0, xr1,
             pay0, pay1, ds0, ds1, ptmp, wtmp, acc,
             gx0, gx1, gr0, gr1, ss0, ss1):
    c = lax.axis_index("c")
    s = lax.axis_index("s")
    wid = c * NS + s
    d = xl0.shape[1]
    nk = d // LN
    xl_b = (xl0, xl1)
    xr_b = (xr0, xr1)
    pay_b = (pay0, pay1)
    ds_b = (ds0, ds1)
    gx = (gx0, gx1)
    gr = (gr0, gr1)
    ss = (ss0, ss1)

    pltpu.sync_copy(mh_hbm, mh_v)
    pltpu.sync_copy(att_hbm, att_v)
    pltpu.sync_copy(ei_hbm.at[0, wid], src_all)
    pltpu.sync_copy(ei_hbm.at[1, wid], dst_all)

    zeros16 = jnp.zeros((LN,), jnp.float32)

    def zero_row(ref, r, ncol):
        for j in range(ncol // LN):
            ref[r, pl.ds(j * LN, LN)] = zeros16

    iota16 = lax.iota(jnp.int32, LN)
    col64 = jnp.full((LN,), d, jnp.int32)
    for pv in pay_b:
        def pz_loop(r, _, pv=pv):
            zero_row(pv, r, PW)
            return 0
        lax.fori_loop(0, EB, pz_loop, 0)

    full = rps // EB
    tail = rps - full * EB
    for j in range(full):
        pltpu.sync_copy(pay0, acc.at[pl.ds(s * rps + j * EB, EB)])
    if tail:
        pltpu.sync_copy(pay0.at[pl.ds(0, tail)],
                        acc.at[pl.ds(s * rps + full * EB, tail)])
    plsc.subcore_barrier()

    att_c = [att_v[pl.ds(k * LN, LN)] for k in range(nk)]
    col_ids = [jnp.full((LN,), j, jnp.int32) for j in range(LN)]

    def issue_gathers(i, slot):
        pltpu.async_copy(xl_hbm.at[src_all.at[i]], xl_b[slot], gx[slot])
        pltpu.async_copy(xr_hbm.at[dst_all.at[i]], xr_b[slot], gr[slot])

    def wait_gathers(slot):
        pltpu.make_async_copy(xl_hbm.at[src_all.at[0]], xl_b[slot], gx[slot]).wait()
        pltpu.make_async_copy(xr_hbm.at[dst_all.at[0]], xr_b[slot], gr[slot]).wait()

    def wait_scatter(slot):
        pltpu.make_async_copy(pay_b[slot], acc.at[ds_b[slot]], ss[slot]).wait()

    def compute_batch(i, slot):
        xl_v = xl_b[slot]
        xr_v = xr_b[slot]
        pay_v = pay_b[slot]

        for g in range(EB // LN):
            dstv = dst_all[i, pl.ds(g * LN, LN)]
            mh = plsc.load_gather(mh_v, [dstv])
            for e in range(LN):
                row = g * LN + e
                p = None
                for k in range(nk):
                    a = xl_v[row, pl.ds(k * LN, LN)]
                    b = xr_v[row, pl.ds(k * LN, LN)]
                    v = a + b
                    lr = jnp.maximum(v, 0.2 * v)
                    t = lr * att_c[k]
                    p = t if p is None else p + t
                ptmp[g, e, pl.ds(0, LN)] = p
            evec = None
            for j in range(LN):
                cj = plsc.load_gather(ptmp.at[g], [iota16, col_ids[j]])
                evec = cj if evec is None else evec + cj
            w = jnp.exp(evec - mh)
            wtmp[g, :] = w
            for e in range(LN):
                row = g * LN + e
                ws = plsc.load_gather(wtmp.at[g], [col_ids[e]])
                for k in range(nk):
                    pay_v[row, pl.ds(k * LN, LN)] = ws * xl_v[row, pl.ds(k * LN, LN)]
            plsc.store_scatter(pay_v, [g * LN + iota16, col64], w)

    def do_batch(i, slot):
        wait_gathers(slot)

        @pl.when(i >= 2)
        def _():
            wait_scatter(slot)

        compute_batch(i, slot)
        for j in range(EB // LN):
            ds_b[slot][pl.ds(j * LN, LN)] = dst_all[i, pl.ds(j * LN, LN)]
        pltpu.async_copy(pay_b[slot], acc.at[ds_b[slot]], ss[slot], add=True)

        @pl.when(i + 2 < nb)
        def _():
            issue_gathers(i + 2, slot)

    issue_gathers(0, 0)
    issue_gathers(1, 1)

    def pair(i2, _):
        do_batch(i2 * 2, 0)
        do_batch(i2 * 2 + 1, 1)
        return 0

    lax.fori_loop(0, nb // 2, pair, 0)
    if nb % 2:
        do_batch(nb - 1, 0)
    wait_scatter((nb - 1) % 2)
    wait_scatter((nb - 2) % 2)
    plsc.subcore_barrier()
    pltpu.sync_copy(acc.at[pl.ds(s * rps, rps)],
                    out_hbm.at[c, pl.ds(s * rps, rps)])


def kernel(x, edge_index, W_l, W_r, att, bias_conv, W_lin, b_lin):
    n, d_in = x.shape
    d = W_l.shape[1]
    e_tot = edge_index.shape[1]

    ei = edge_index.astype(jnp.int32)
    src, dst = ei[0], ei[1]
    wcat = jnp.concatenate([W_l, W_r], axis=1)

    xl, xr, mh2 = pl.pallas_call(
        _tc1_body,
        out_shape=(
            jax.ShapeDtypeStruct((n, d), jnp.float32),
            jax.ShapeDtypeStruct((n, d), jnp.float32),
            jax.ShapeDtypeStruct((n, 1), jnp.float32),
        ),
    )(x, wcat, att.reshape(d, 1))
    mh = mh2.reshape(n)

    chunk = NW * EB
    e_pad = ((e_tot + chunk - 1) // chunk) * chunk
    n_acc = ((n + 8 + 127) // 128) * 128
    if e_pad != e_tot:
        pad = e_pad - e_tot
        src = jnp.concatenate([src, jnp.zeros((pad,), jnp.int32)])
        dst = jnp.concatenate([dst, jnp.full((pad,), n, jnp.int32)])
        ei = jnp.stack([src, dst])

    ew = e_pad // NW
    nb = ew // EB
    rps = n_acc // NS
    ei4 = ei.reshape(2, NW, nb, EB)

    mesh = plsc.VectorSubcoreMesh(core_axis_name="c", subcore_axis_name="s")
    sc_call = pl.kernel(
        functools.partial(_sc_body, nb, rps),
        out_type=jax.ShapeDtypeStruct((NC, n_acc, PW), jnp.float32),
        mesh=mesh,
        compiler_params=pltpu.CompilerParams(
            needs_layout_passes=False, use_tc_tiling_on_sc=False),
        scratch_types=[
            pltpu.VMEM((n,), jnp.float32),
            pltpu.VMEM((d,), jnp.float32),
            pltpu.VMEM((nb, EB), jnp.int32),
            pltpu.VMEM((nb, EB), jnp.int32),
            pltpu.VMEM((EB, d), jnp.float32),
            pltpu.VMEM((EB, d), jnp.float32),
            pltpu.VMEM((EB, d), jnp.float32),
            pltpu.VMEM((EB, d), jnp.float32),
            pltpu.VMEM((EB, PW), jnp.float32),
            pltpu.VMEM((EB, PW), jnp.float32),
            pltpu.VMEM((EB,), jnp.int32),
            pltpu.VMEM((EB,), jnp.int32),
            pltpu.VMEM((EB // LN, LN, LN + 1), jnp.float32),
            pltpu.VMEM((EB // LN, LN), jnp.float32),
            pltpu.VMEM_SHARED((n_acc, PW), jnp.float32),
            pltpu.SemaphoreType.DMA,
            pltpu.SemaphoreType.DMA,
            pltpu.SemaphoreType.DMA,
            pltpu.SemaphoreType.DMA,
            pltpu.SemaphoreType.DMA,
            pltpu.SemaphoreType.DMA,
        ],
    )
    acc = sc_call(ei4, xl, xr, mh, att)

    out = pl.pallas_call(
        _tc2_body,
        out_shape=jax.ShapeDtypeStruct((n, 1), jnp.float32),
    )(acc, xl, bias_conv.reshape(1, d), W_lin, b_lin.reshape(1, 1))
    return out


# speedup vs baseline: 1.2266x; 1.2266x over previous
import functools

import jax
import jax.numpy as jnp
from jax import lax
from jax.experimental import pallas as pl
from jax.experimental.pallas import tpu as pltpu
from jax.experimental.pallas import tpu_sc as plsc

NC = 2
NS = 16
NW = NC * NS
LN = 16
EB = 80
PW = 80


def _tc1_body(x_ref, wcat_ref, att_ref, xl_ref, xlr_ref, mh_ref):
    y = jnp.dot(x_ref[...], wcat_ref[...], preferred_element_type=jnp.float32)
    d = y.shape[1] // 2
    xl = y[:, :d]
    xr = y[:, d:]
    xl_ref[...] = xl
    xlr_ref[...] = y.astype(jnp.bfloat16)
    s = xl + xr
    lr = jnp.where(s >= 0, s, 0.2 * s)
    mh_ref[...] = jnp.dot(lr, att_ref[...], preferred_element_type=jnp.float32)


def _tc2_body(acc_ref, xl_ref, bias_ref, wlin_ref, blin_ref, out_ref):
    n = xl_ref.shape[0]
    s = acc_ref[0][:n] + acc_ref[1][:n]
    d = xl_ref.shape[1]
    h = s[:, :d] + xl_ref[...]
    den = s[:, d:d + 1] + 1.0
    h = h / den + bias_ref[...]
    h = jnp.maximum(h, 0.0)
    z = jnp.dot(h, wlin_ref[...], preferred_element_type=jnp.float32)
    out_ref[...] = jax.nn.sigmoid(z + blin_ref[...])


def _sc_body(nb, rps,
             ei_hbm, xlr_hbm, mh_hbm, att_hbm, out_hbm,
             mh_v, att_v, src_all, dst_all, sb0, sb1, db0, db1,
             pay0, pay1, ds0, ds1, ptmp, wtmp, acc,
             gx0, gx1, gr0, gr1, ss0, ss1):
    c = lax.axis_index("c")
    s = lax.axis_index("s")
    wid = c * NS + s
    d = sb0.shape[1] // 2
    sb_b = (sb0, sb1)
    db_b = (db0, db1)
    pay_b = (pay0, pay1)
    ds_b = (ds0, ds1)
    gx = (gx0, gx1)
    gr = (gr0, gr1)
    ss = (ss0, ss1)

    pltpu.sync_copy(mh_hbm, mh_v)
    pltpu.sync_copy(att_hbm, att_v)
    pltpu.sync_copy(ei_hbm.at[0, wid], src_all)
    pltpu.sync_copy(ei_hbm.at[1, wid], dst_all)

    zeros16 = jnp.zeros((LN,), jnp.float32)

    def zero_row(ref, r, ncol):
        for j in range(ncol // LN):
            ref[r, pl.ds(j * LN, LN)] = zeros16

    iota16 = lax.iota(jnp.int32, LN)
    col64 = jnp.full((LN,), d, jnp.int32)
    for pv in pay_b:
        def pz_loop(r, _, pv=pv):
            zero_row(pv, r, PW)
            return 0
        lax.fori_loop(0, EB, pz_loop, 0)

    full = rps // EB
    tail = rps - full * EB
    for j in range(full):
        pltpu.sync_copy(pay0, acc.at[pl.ds(s * rps + j * EB, EB)])
    if tail:
        pltpu.sync_copy(pay0.at[pl.ds(0, tail)],
                        acc.at[pl.ds(s * rps + full * EB, tail)])
    plsc.subcore_barrier()

    nc2 = d // 32
    att_c = [att_v[pl.ds(k * 32, 32)] for k in range(nc2)]
    col_ids = [jnp.full((LN,), j, jnp.int32) for j in range(LN)]
    himask = jnp.full((LN,), -65536, jnp.int32)
    neg_slope = jnp.bfloat16(0.2)

    def unpack2(x32):
        raw = plsc.bitcast(x32, jnp.int32)
        ev = plsc.bitcast(lax.shift_left(raw, 16), jnp.float32)
        od = plsc.bitcast(lax.bitwise_and(raw, himask), jnp.float32)
        return ev, od

    def issue_gathers(i, slot):
        pltpu.async_copy(xlr_hbm.at[src_all.at[i]], sb_b[slot], gx[slot])
        pltpu.async_copy(xlr_hbm.at[dst_all.at[i]], db_b[slot], gr[slot])

    def wait_gathers(slot):
        pltpu.make_async_copy(xlr_hbm.at[src_all.at[0]], sb_b[slot], gx[slot]).wait()
        pltpu.make_async_copy(xlr_hbm.at[dst_all.at[0]], db_b[slot], gr[slot]).wait()

    def wait_scatter(slot):
        pltpu.make_async_copy(pay_b[slot], acc.at[ds_b[slot]], ss[slot]).wait()

    def compute_batch(i, slot):
        sb_v = sb_b[slot]
        db_v = db_b[slot]
        pay_v = pay_b[slot]

        for g in range(EB // LN):
            dstv = dst_all[i, pl.ds(g * LN, LN)]
            mh = plsc.load_gather(mh_v, [dstv])
            for e in range(LN):
                row = g * LN + e
                p32 = None
                for k in range(nc2):
                    a = sb_v[row, pl.ds(k * 32, 32)]
                    b = db_v[row, pl.ds(d + k * 32, 32)]
                    v = a + b
                    lr = jnp.maximum(v, neg_slope * v)
                    t = lr * att_c[k]
                    p32 = t if p32 is None else p32 + t
                ev, od = unpack2(p32)
                ptmp[g, e, pl.ds(0, LN)] = ev + od
            evec = None
            for j in range(LN):
                cj = plsc.load_gather(ptmp.at[g], [iota16, col_ids[j]])
                evec = cj if evec is None else evec + cj
            w = jnp.exp(evec - mh)
            wtmp[g, :] = w
            for e in range(LN):
                row = g * LN + e
                ws = plsc.load_gather(wtmp.at[g], [col_ids[e]])
                for k in range(nc2):
                    ev, od = unpack2(sb_v[row, pl.ds(k * 32, 32)])
                    pay_v[row, pl.ds(k * 32, LN)] = ws * ev
                    pay_v[row, pl.ds(k * 32 + LN, LN)] = ws * od
            plsc.store_scatter(pay_v, [g * LN + iota16, col64], w)

    def do_batch(i, slot):
        wait_gathers(slot)

        @pl.when(i >= 2)
        def _():
            wait_scatter(slot)

        compute_batch(i, slot)
        for j in range(EB // LN):
            ds_b[slot][pl.ds(j * LN, LN)] = dst_all[i, pl.ds(j * LN, LN)]
        pltpu.async_copy(pay_b[slot], acc.at[ds_b[slot]], ss[slot], add=True)

        @pl.when(i + 2 < nb)
        def _():
            issue_gathers(i + 2, slot)

    issue_gathers(0, 0)
    issue_gathers(1, 1)

    def pair(i2, _):
        do_batch(i2 * 2, 0)
        do_batch(i2 * 2 + 1, 1)
        return 0

    lax.fori_loop(0, nb // 2, pair, 0)
    if nb % 2:
        do_batch(nb - 1, 0)
    wait_scatter((nb - 1) % 2)
    wait_scatter((nb - 2) % 2)
    plsc.subcore_barrier()
    pltpu.sync_copy(acc.at[pl.ds(s * rps, rps)],
                    out_hbm.at[c, pl.ds(s * rps, rps)])


def kernel(x, edge_index, W_l, W_r, att, bias_conv, W_lin, b_lin):
    n, d_in = x.shape
    d = W_l.shape[1]
    e_tot = edge_index.shape[1]

    ei = edge_index.astype(jnp.int32)
    src, dst = ei[0], ei[1]
    wcat = jnp.concatenate([W_l, W_r], axis=1)

    xl, xlr_bf, mh2 = pl.pallas_call(
        _tc1_body,
        out_shape=(
            jax.ShapeDtypeStruct((n, d), jnp.float32),
            jax.ShapeDtypeStruct((n, 2 * d), jnp.bfloat16),
            jax.ShapeDtypeStruct((n, 1), jnp.float32),
        ),
    )(x, wcat, att.reshape(d, 1))
    mh = mh2.reshape(n)

    chunk = NW * EB
    e_pad = ((e_tot + chunk - 1) // chunk) * chunk
    n_acc = ((n + 8 + 127) // 128) * 128
    if e_pad != e_tot:
        pad = e_pad - e_tot
        src = jnp.concatenate([src, jnp.zeros((pad,), jnp.int32)])
        dst = jnp.concatenate([dst, jnp.full((pad,), n, jnp.int32)])
        ei = jnp.stack([src, dst])

    ew = e_pad // NW
    nb = ew // EB
    rps = n_acc // NS
    ei4 = ei.reshape(2, NW, nb, EB)

    mesh = plsc.VectorSubcoreMesh(core_axis_name="c", subcore_axis_name="s")
    sc_call = pl.kernel(
        functools.partial(_sc_body, nb, rps),
        out_type=jax.ShapeDtypeStruct((NC, n_acc, PW), jnp.float32),
        mesh=mesh,
        compiler_params=pltpu.CompilerParams(
            needs_layout_passes=False, use_tc_tiling_on_sc=False),
        scratch_types=[
            pltpu.VMEM((n,), jnp.float32),
            pltpu.VMEM((d,), jnp.bfloat16),
            pltpu.VMEM((nb, EB), jnp.int32),
            pltpu.VMEM((nb, EB), jnp.int32),
            pltpu.VMEM((EB, 2 * d), jnp.bfloat16),
            pltpu.VMEM((EB, 2 * d), jnp.bfloat16),
            pltpu.VMEM((EB, 2 * d), jnp.bfloat16),
            pltpu.VMEM((EB, 2 * d), jnp.bfloat16),
            pltpu.VMEM((EB, PW), jnp.float32),
            pltpu.VMEM((EB, PW), jnp.float32),
            pltpu.VMEM((EB,), jnp.int32),
            pltpu.VMEM((EB,), jnp.int32),
            pltpu.VMEM((EB // LN, LN, LN + 1), jnp.float32),
            pltpu.VMEM((EB // LN, LN), jnp.float32),
            pltpu.VMEM_SHARED((n_acc, PW), jnp.float32),
            pltpu.SemaphoreType.DMA,
            pltpu.SemaphoreType.DMA,
            pltpu.SemaphoreType.DMA,
            pltpu.SemaphoreType.DMA,
            pltpu.SemaphoreType.DMA,
            pltpu.SemaphoreType.DMA,
        ],
    )
    acc = sc_call(ei4, xlr_bf, mh, att.astype(jnp.bfloat16))

    sigma = []
    for blk in range(d // 32):
        sigma += [32 * blk + 2 * j for j in range(16)]
        sigma += [32 * blk + 2 * j + 1 for j in range(16)]
    sigma = jnp.asarray(sigma, jnp.int32)
    out = pl.pallas_call(
        _tc2_body,
        out_shape=jax.ShapeDtypeStruct((n, 1), jnp.float32),
    )(acc, xl[:, sigma], bias_conv[sigma].reshape(1, d), W_lin[sigma],
      b_lin.reshape(1, 1))
    return out


# speedup vs baseline: 1.5692x; 1.2793x over previous
import functools

import jax
import jax.numpy as jnp
from jax import lax
from jax.experimental import pallas as pl
from jax.experimental.pallas import tpu as pltpu
from jax.experimental.pallas import tpu_sc as plsc

NC = 2
NS = 16
NW = NC * NS
LN = 16
EB = 80
PW = 80


def _tc1_body(x_ref, wcat_ref, att_ref, xlr_ref, mh_ref):
    y = jnp.dot(x_ref[...], wcat_ref[...], preferred_element_type=jnp.float32)
    d = y.shape[1] // 2
    xl = y[:, :d]
    xr = y[:, d:]
    xlr_ref[...] = y.astype(jnp.bfloat16)
    s = xl + xr
    lr = jnp.where(s >= 0, s, 0.2 * s)
    mh_ref[...] = jnp.dot(lr, att_ref[...], preferred_element_type=jnp.float32)


def _tc2_body(acc_ref, x_ref, wlsig_ref, bias_ref, wlin_ref, blin_ref,
              out_ref):
    n = x_ref.shape[0]
    d = wlsig_ref.shape[1]
    xl = jnp.dot(x_ref[...], wlsig_ref[...], preferred_element_type=jnp.float32)
    s = acc_ref[0][:n] + acc_ref[1][:n]
    h = s[:, :d] + xl
    den = s[:, d:d + 1] + 1.0
    h = h / den + bias_ref[...]
    h = jnp.maximum(h, 0.0)
    z = jnp.dot(h, wlin_ref[...], preferred_element_type=jnp.float32)
    out_ref[...] = jax.nn.sigmoid(z + blin_ref[...])


def _sc_body(nb, rps,
             ei_hbm, xlr_hbm, mh_hbm, att_hbm, out_hbm,
             mh_v, att_v, src_all, dst_all, sb0, sb1, db0, db1,
             pay0, pay1, ds0, ds1, ptmp, acc,
             gx0, gx1, gr0, gr1, ss0, ss1):
    c = lax.axis_index("c")
    s = lax.axis_index("s")
    wid = c * NS + s
    d = sb0.shape[1] // 2
    sb_b = (sb0, sb1)
    db_b = (db0, db1)
    pay_b = (pay0, pay1)
    ds_b = (ds0, ds1)
    gx = (gx0, gx1)
    gr = (gr0, gr1)
    ss = (ss0, ss1)

    pltpu.sync_copy(mh_hbm, mh_v)
    pltpu.sync_copy(att_hbm, att_v)
    pltpu.sync_copy(ei_hbm.at[0, wid], src_all)
    pltpu.sync_copy(ei_hbm.at[1, wid], dst_all)

    zeros16 = jnp.zeros((LN,), jnp.float32)

    def zero_row(ref, r, ncol):
        for j in range(ncol // LN):
            ref[r, pl.ds(j * LN, LN)] = zeros16

    iota16 = lax.iota(jnp.int32, LN)
    col64 = jnp.full((LN,), d, jnp.int32)
    for pv in pay_b:
        def pz_loop(r, _, pv=pv):
            zero_row(pv, r, PW)
            return 0
        lax.fori_loop(0, EB, pz_loop, 0)

    full = rps // EB
    tail = rps - full * EB
    for j in range(full):
        pltpu.sync_copy(pay0, acc.at[pl.ds(s * rps + j * EB, EB)])
    if tail:
        pltpu.sync_copy(pay0.at[pl.ds(0, tail)],
                        acc.at[pl.ds(s * rps + full * EB, tail)])
    plsc.subcore_barrier()

    nc2 = d // 32
    att_c = [att_v[pl.ds(k * 32, 32)] for k in range(nc2)]
    col_ids = [jnp.full((LN,), j, jnp.int32) for j in range(LN)]
    himask = jnp.full((LN,), -65536, jnp.int32)
    neg_slope = jnp.bfloat16(0.2)

    def unpack2(x32):
        raw = plsc.bitcast(x32, jnp.int32)
        ev = plsc.bitcast(lax.shift_left(raw, 16), jnp.float32)
        od = plsc.bitcast(lax.bitwise_and(raw, himask), jnp.float32)
        return ev, od

    def issue_gathers(i, slot):
        pltpu.async_copy(xlr_hbm.at[src_all.at[i]], sb_b[slot], gx[slot])
        pltpu.async_copy(xlr_hbm.at[dst_all.at[i]], db_b[slot], gr[slot])

    def wait_gathers(slot):
        pltpu.make_async_copy(xlr_hbm.at[src_all.at[0]], sb_b[slot], gx[slot]).wait()
        pltpu.make_async_copy(xlr_hbm.at[dst_all.at[0]], db_b[slot], gr[slot]).wait()

    def wait_scatter(slot):
        pltpu.make_async_copy(pay_b[slot], acc.at[ds_b[slot]], ss[slot]).wait()

    def compute_batch(i, slot):
        sb_v = sb_b[slot]
        db_v = db_b[slot]
        pay_v = pay_b[slot]

        for g in range(EB // LN):
            dstv = dst_all[i, pl.ds(g * LN, LN)]
            mh = plsc.load_gather(mh_v, [dstv])
            for e in range(LN):
                row = g * LN + e
                p32 = None
                for k in range(nc2):
                    a = sb_v[row, pl.ds(k * 32, 32)]
                    b = db_v[row, pl.ds(d + k * 32, 32)]
                    v = a + b
                    lr = jnp.maximum(v, neg_slope * v)
                    t = lr * att_c[k]
                    p32 = t if p32 is None else p32 + t
                ev, od = unpack2(p32)
                ptmp[g, e, pl.ds(0, LN)] = ev + od
            evec = None
            for j in range(LN):
                cj = plsc.load_gather(ptmp.at[g], [iota16, col_ids[j]])
                evec = cj if evec is None else evec + cj
            w = jnp.exp(evec - mh)
            for e in range(LN):
                row = g * LN + e
                ws = lax.broadcast_in_dim(
                    jnp.sum(jnp.where(iota16 == e, w, 0.0)), (LN,), ())
                for k in range(nc2):
                    ev, od = unpack2(sb_v[row, pl.ds(k * 32, 32)])
                    pay_v[row, pl.ds(k * 32, LN)] = ws * ev
                    pay_v[row, pl.ds(k * 32 + LN, LN)] = ws * od
            plsc.store_scatter(pay_v, [g * LN + iota16, col64], w)

    def do_batch(i, slot):
        wait_gathers(slot)

        @pl.when(i >= 2)
        def _():
            wait_scatter(slot)

        compute_batch(i, slot)
        for j in range(EB // LN):
            ds_b[slot][pl.ds(j * LN, LN)] = dst_all[i, pl.ds(j * LN, LN)]
        pltpu.async_copy(pay_b[slot], acc.at[ds_b[slot]], ss[slot], add=True)

        @pl.when(i + 2 < nb)
        def _():
            issue_gathers(i + 2, slot)

    issue_gathers(0, 0)
    issue_gathers(1, 1)

    def pair(i2, _):
        do_batch(i2 * 2, 0)
        do_batch(i2 * 2 + 1, 1)
        return 0

    lax.fori_loop(0, nb // 2, pair, 0)
    if nb % 2:
        do_batch(nb - 1, 0)
    wait_scatter((nb - 1) % 2)
    wait_scatter((nb - 2) % 2)
    plsc.subcore_barrier()
    pltpu.sync_copy(acc.at[pl.ds(s * rps, rps)],
                    out_hbm.at[c, pl.ds(s * rps, rps)])


def kernel(x, edge_index, W_l, W_r, att, bias_conv, W_lin, b_lin):
    n, d_in = x.shape
    d = W_l.shape[1]
    e_tot = edge_index.shape[1]

    ei = edge_index.astype(jnp.int32)
    src, dst = ei[0], ei[1]
    wcat = jnp.concatenate([W_l, W_r], axis=1)

    xlr_bf, mh2 = pl.pallas_call(
        _tc1_body,
        out_shape=(
            jax.ShapeDtypeStruct((n, 2 * d), jnp.bfloat16),
            jax.ShapeDtypeStruct((n, 1), jnp.float32),
        ),
    )(x, wcat, att.reshape(d, 1))
    mh = mh2.reshape(n)

    chunk = NW * EB
    e_pad = ((e_tot + chunk - 1) // chunk) * chunk
    n_acc = ((n + 8 + 127) // 128) * 128
    if e_pad != e_tot:
        pad = e_pad - e_tot
        src = jnp.concatenate([src, jnp.zeros((pad,), jnp.int32)])
        dst = jnp.concatenate([dst, jnp.full((pad,), n, jnp.int32)])
        ei = jnp.stack([src, dst])

    ew = e_pad // NW
    nb = ew // EB
    rps = n_acc // NS
    ei4 = ei.reshape(2, NW, nb, EB)

    mesh = plsc.VectorSubcoreMesh(core_axis_name="c", subcore_axis_name="s")
    sc_call = pl.kernel(
        functools.partial(_sc_body, nb, rps),
        out_type=jax.ShapeDtypeStruct((NC, n_acc, PW), jnp.float32),
        mesh=mesh,
        compiler_params=pltpu.CompilerParams(
            needs_layout_passes=False, use_tc_tiling_on_sc=False),
        scratch_types=[
            pltpu.VMEM((n,), jnp.float32),
            pltpu.VMEM((d,), jnp.bfloat16),
            pltpu.VMEM((nb, EB), jnp.int32),
            pltpu.VMEM((nb, EB), jnp.int32),
            pltpu.VMEM((EB, 2 * d), jnp.bfloat16),
            pltpu.VMEM((EB, 2 * d), jnp.bfloat16),
            pltpu.VMEM((EB, 2 * d), jnp.bfloat16),
            pltpu.VMEM((EB, 2 * d), jnp.bfloat16),
            pltpu.VMEM((EB, PW), jnp.float32),
            pltpu.VMEM((EB, PW), jnp.float32),
            pltpu.VMEM((EB,), jnp.int32),
            pltpu.VMEM((EB,), jnp.int32),
            pltpu.VMEM((EB // LN, LN, LN + 1), jnp.float32),
            pltpu.VMEM_SHARED((n_acc, PW), jnp.float32),
            pltpu.SemaphoreType.DMA,
            pltpu.SemaphoreType.DMA,
            pltpu.SemaphoreType.DMA,
            pltpu.SemaphoreType.DMA,
            pltpu.SemaphoreType.DMA,
            pltpu.SemaphoreType.DMA,
        ],
    )
    acc = sc_call(ei4, xlr_bf, mh, att.astype(jnp.bfloat16))

    sigma = []
    for blk in range(d // 32):
        sigma += [32 * blk + 2 * j for j in range(16)]
        sigma += [32 * blk + 2 * j + 1 for j in range(16)]
    out = pl.pallas_call(
        _tc2_body,
        out_shape=jax.ShapeDtypeStruct((n, 1), jnp.float32),
    )(acc, x, W_l[:, sigma], bias_conv[jnp.asarray(sigma)].reshape(1, d),
      W_lin[jnp.asarray(sigma)], b_lin.reshape(1, 1))
    return out
